# baseline (speedup 1.0000x reference)
import jax
import jax.numpy as jnp
from jax import lax
from jax.experimental import pallas as pl
from jax.experimental.pallas import tpu as pltpu
from jax.experimental.pallas import tpu_sc as plsc

_NC = 2
_NS = 16
_L = 16
_NW = _NC * _NS
_N_CHUNKS = 2
_BM = 4096


def _tc_logits_body(x_ref, wt_ref, b_ref, out_ref):
    p = jnp.dot(x_ref[...], wt_ref[...], preferred_element_type=jnp.float32)
    out_ref[...] = p.T + b_ref[...]


def _make_sc_top1(n_experts, tpw):
    def body(lt_hbm, score_hbm, idx_hbm, lbuf, sbuf, ibuf):
        wid = lax.axis_index("s") * _NC + lax.axis_index("c")
        pltpu.sync_copy(lt_hbm.at[:, pl.ds(wid * tpw, tpw)], lbuf)

        def step(j, _):
            off = j * _L
            ls = [lbuf[e, pl.ds(off, _L)] for e in range(n_experts)]
            m = ls[0]
            idx = jnp.zeros((_L,), jnp.int32)
            for e in range(1, n_experts):
                g = ls[e] > m
                m = jnp.where(g, ls[e], m)
                idx = jnp.where(g, jnp.full((_L,), e, jnp.int32), idx)
            s = jnp.exp(ls[0] - m)
            for e in range(1, n_experts):
                s = s + jnp.exp(ls[e] - m)
            sbuf[pl.ds(off, _L)] = 1.0 / s
            ibuf[pl.ds(off, _L)] = idx
            return 0

        lax.fori_loop(0, tpw // _L, step, 0)
        base = wid * tpw
        pltpu.sync_copy(sbuf, score_hbm.at[pl.ds(base, tpw)])
        pltpu.sync_copy(ibuf, idx_hbm.at[pl.ds(base, tpw)])

    return body


def kernel(x, W, b):
    d_model = x.shape[-1]
    n_experts = W.shape[0]
    x_flat = x.reshape(-1, d_model)
    n_tok = x_flat.shape[0]
    chunk = n_tok // _N_CHUNKS
    tpw = chunk // _NW
    blk_per_chunk = chunk // _BM

    mesh = plsc.VectorSubcoreMesh(core_axis_name="c", subcore_axis_name="s")
    sc_top1 = pl.kernel(
        _make_sc_top1(n_experts, tpw),
        out_type=(
            jax.ShapeDtypeStruct((chunk,), jnp.float32),
            jax.ShapeDtypeStruct((chunk,), jnp.int32),
        ),
        mesh=mesh,
        scratch_types=[
            pltpu.VMEM((n_experts, tpw), jnp.float32),
            pltpu.VMEM((tpw,), jnp.float32),
            pltpu.VMEM((tpw,), jnp.int32),
        ],
    )

    wt = W.T
    b_col = b.reshape(n_experts, 1)
    scores, idxs = [], []
    for ci in range(_N_CHUNKS):
        base_blk = ci * blk_per_chunk
        logits_t = pl.pallas_call(
            _tc_logits_body,
            grid=(blk_per_chunk,),
            in_specs=[
                pl.BlockSpec((_BM, d_model),
                             lambda i, bb=base_blk: (i + bb, 0)),
                pl.BlockSpec((d_model, n_experts), lambda i: (0, 0)),
                pl.BlockSpec((n_experts, 1), lambda i: (0, 0)),
            ],
            out_specs=pl.BlockSpec((n_experts, _BM), lambda i: (0, i)),
            out_shape=jax.ShapeDtypeStruct((n_experts, chunk), jnp.float32),
        )(x_flat, wt, b_col)
        s, i = sc_top1(logits_t)
        scores.append(s.reshape(chunk, 1))
        idxs.append(i.reshape(chunk, 1))

    return (jnp.concatenate(scores, axis=0), jnp.concatenate(idxs, axis=0))
